# baseline (speedup 1.0000x reference)
import functools

import jax
import jax.numpy as jnp
from jax import lax
from jax.experimental import pallas as pl
from jax.experimental.pallas import tpu as pltpu
from jax.experimental.pallas import tpu_sc as plsc

NC = 2
NS = 16
CHUNK = 80
NBUF = 3


def _sc_partials(n_nodes, n_edges, d_feat):
    e_per_w = n_edges // (NC * NS)
    n_chunks = e_per_w // CHUNK
    rpt = (n_nodes // NS) // 8 * 8
    rem = n_nodes - NS * rpt
    n_main = (n_chunks // NBUF) * NBUF
    mesh = plsc.VectorSubcoreMesh(
        core_axis_name="c", subcore_axis_name="s", num_cores=NC, num_subcores=NS
    )

    @functools.partial(
        pl.kernel,
        out_type=jax.ShapeDtypeStruct((NC * n_nodes, d_feat), jnp.float32),
        mesh=mesh,
        scratch_types=[
            pltpu.VMEM((n_chunks, CHUNK), jnp.int32),
            [pltpu.VMEM((CHUNK, d_feat), jnp.float32) for _ in range(NBUF)],
            [pltpu.SemaphoreType.DMA for _ in range(NBUF)],
            [pltpu.SemaphoreType.DMA for _ in range(NBUF)],
            pltpu.SemaphoreType.DMA,
            pltpu.VMEM_SHARED((n_nodes, d_feat), jnp.float32),
        ],
    )
    def scatter_add(
        emb_hbm, dst_hbm, zeros_hbm, out_hbm, idx_all, rows, gsems, ssems, zsem, acc
    ):
        c = lax.axis_index("c")
        s = lax.axis_index("s")
        wid = s * NC + c
        edge_base = wid * e_per_w

        def issue(j, b):
            pltpu.async_copy(
                emb_hbm.at[pl.ds(edge_base + j * CHUNK, CHUNK)], rows[b], gsems[b]
            )

        def wait_gather(b):
            pltpu.make_async_copy(
                emb_hbm.at[pl.ds(0, CHUNK)], rows[b], gsems[b]
            ).wait()

        def scat(j, b):
            pass

        def wait_scat(b):
            pass

        icp = pltpu.async_copy(dst_hbm.at[1, wid], idx_all, zsem)
        r0 = s * rpt
        zcp = pltpu.async_copy(
            zeros_hbm.at[pl.ds(r0, rpt)], acc.at[pl.ds(r0, rpt)], zsem
        )
        if rem:
            @pl.when(s == 0)
            def _():
                pltpu.async_copy(
                    zeros_hbm.at[pl.ds(NS * rpt, rem)],
                    acc.at[pl.ds(NS * rpt, rem)],
                    zsem,
                ).wait()
        for b in range(NBUF):
            issue(b, b)
        icp.wait()
        zcp.wait()
        plsc.subcore_barrier()

        def body(g, _):
            for b in range(NBUF):
                j = g * NBUF + b
                wait_gather(b)
                scat(j, b)
                pb = (b - 1) % NBUF

                def retire_prev():
                    wait_scat(pb)
                    nxt = j - 1 + NBUF

                    @pl.when(nxt < n_chunks)
                    def _():
                        issue(nxt, pb)

                if b == 0:
                    pl.when(g > 0)(retire_prev)
                else:
                    retire_prev()
            return 0

        lax.fori_loop(0, n_main // NBUF, body, 0)
        for j in range(n_main, n_chunks):
            b = j % NBUF
            wait_gather(b)
            scat(j, b)
        for j in range(max(n_main - 1, 0), n_chunks):
            wait_scat(j % NBUF)
        plsc.subcore_barrier()

        pltpu.sync_copy(
            acc.at[pl.ds(r0, rpt)],
            out_hbm.at[pl.ds(c * n_nodes + r0, rpt)],
        )
        if rem:
            @pl.when(s == 0)
            def _():
                pltpu.sync_copy(
                    acc.at[pl.ds(NS * rpt, rem)],
                    out_hbm.at[pl.ds(c * n_nodes + NS * rpt, rem)],
                )

    return scatter_add


def _add_body(a_ref, b_ref, o_ref):
    o_ref[...] = a_ref[...] + b_ref[...]


def _combine(partials, n_nodes, d_feat):
    rows = 1000
    n_blk = n_nodes // rows
    return pl.pallas_call(
        _add_body,
        grid=(n_blk,),
        in_specs=[
            pl.BlockSpec((rows, d_feat), lambda i: (i, 0)),
            pl.BlockSpec((rows, d_feat), lambda i: (i + n_blk, 0)),
        ],
        out_specs=pl.BlockSpec((rows, d_feat), lambda i: (i, 0)),
        out_shape=jax.ShapeDtypeStruct((n_nodes, d_feat), jnp.float32),
    )(partials, partials)


def kernel(src_emb, edge_index, src_emb_in):
    n_edges, d_feat = src_emb.shape
    n_nodes = src_emb_in.shape[0]
    e_per_w = n_edges // (NC * NS)
    dst4 = edge_index.astype(jnp.int32).reshape(
        2, NC * NS, e_per_w // CHUNK, CHUNK
    )
    zeros = jnp.zeros((n_nodes, d_feat), jnp.float32)
    partials = _sc_partials(n_nodes, n_edges, d_feat)(src_emb, dst4, zeros)
    return _combine(partials, n_nodes, d_feat)


# speedup vs baseline: 1.0497x; 1.0497x over previous
import functools

import jax
import jax.numpy as jnp
from jax import lax
from jax.experimental import pallas as pl
from jax.experimental.pallas import tpu as pltpu
from jax.experimental.pallas import tpu_sc as plsc

NC = 2
NS = 16
CHUNK = 80
NBUF = 3


def _sc_partials(n_nodes, n_edges, d_feat):
    e_per_w = n_edges // (NC * NS)
    n_chunks = e_per_w // CHUNK
    rpt = (n_nodes // NS) // 8 * 8
    rem = n_nodes - NS * rpt
    n_main = (n_chunks // NBUF) * NBUF
    mesh = plsc.VectorSubcoreMesh(
        core_axis_name="c", subcore_axis_name="s", num_cores=NC, num_subcores=NS
    )

    @functools.partial(
        pl.kernel,
        out_type=jax.ShapeDtypeStruct((NC, n_nodes, d_feat), jnp.float32),
        mesh=mesh,
        scratch_types=[
            pltpu.VMEM((n_chunks, CHUNK), jnp.int32),
            [pltpu.VMEM((CHUNK, d_feat), jnp.float32) for _ in range(NBUF)],
            [pltpu.SemaphoreType.DMA for _ in range(NBUF)],
            [pltpu.SemaphoreType.DMA for _ in range(NBUF)],
            pltpu.SemaphoreType.DMA,
            pltpu.VMEM_SHARED((n_nodes, d_feat), jnp.float32),
        ],
    )
    def scatter_add(
        emb_hbm, dst_hbm, out_hbm, idx_all, rows, gsems, ssems, zsem, acc
    ):
        c = lax.axis_index("c")
        s = lax.axis_index("s")
        wid = s * NC + c
        edge_base = wid * e_per_w

        def issue(j, b):
            pltpu.async_copy(
                emb_hbm.at[pl.ds(edge_base + j * CHUNK, CHUNK)], rows[b], gsems[b]
            )

        def wait_gather(b):
            pltpu.make_async_copy(
                emb_hbm.at[pl.ds(0, CHUNK)], rows[b], gsems[b]
            ).wait()

        def scat(j, b):
            pltpu.async_copy(rows[b], acc.at[idx_all.at[j]], ssems[b], add=True)

        def wait_scat(b):
            pltpu.make_async_copy(rows[b], acc.at[pl.ds(0, CHUNK)], ssems[b]).wait()

        icp = pltpu.async_copy(dst_hbm.at[1, wid], idx_all, zsem)
        r0 = s * rpt
        z16 = jnp.zeros((16,), jnp.float32)

        def zrow(i, _):
            for k in range(d_feat // 16):
                rows[0][i, pl.ds(k * 16, 16)] = z16
            return 0

        lax.fori_loop(0, CHUNK, zrow, 0)
        n_zc = rpt // CHUNK
        z_rem = rpt - n_zc * CHUNK
        for k in range(n_zc):
            pltpu.sync_copy(rows[0], acc.at[pl.ds(r0 + k * CHUNK, CHUNK)])
        if z_rem:
            pltpu.sync_copy(
                rows[0].at[pl.ds(0, z_rem)],
                acc.at[pl.ds(r0 + n_zc * CHUNK, z_rem)],
            )
        if rem:
            @pl.when(s == 0)
            def _():
                pltpu.sync_copy(
                    rows[0].at[pl.ds(0, rem)], acc.at[pl.ds(NS * rpt, rem)]
                )
        for b in range(NBUF):
            issue(b, b)
        icp.wait()
        plsc.subcore_barrier()

        def body(g, _):
            for b in range(NBUF):
                j = g * NBUF + b
                wait_gather(b)
                scat(j, b)
                pb = (b - 1) % NBUF

                def retire_prev():
                    wait_scat(pb)
                    nxt = j - 1 + NBUF

                    @pl.when(nxt < n_chunks)
                    def _():
                        issue(nxt, pb)

                if b == 0:
                    pl.when(g > 0)(retire_prev)
                else:
                    retire_prev()
            return 0

        lax.fori_loop(0, n_main // NBUF, body, 0)
        for j in range(n_main, n_chunks):
            b = j % NBUF
            wait_gather(b)
            scat(j, b)
        for j in range(max(n_main - 1, 0), n_chunks):
            wait_scat(j % NBUF)
        plsc.subcore_barrier()

        pltpu.sync_copy(
            acc.at[pl.ds(r0, rpt)],
            out_hbm.at[c, pl.ds(r0, rpt)],
        )
        if rem:
            @pl.when(s == 0)
            def _():
                pltpu.sync_copy(
                    acc.at[pl.ds(NS * rpt, rem)],
                    out_hbm.at[c, pl.ds(NS * rpt, rem)],
                )

    return scatter_add


def _add_body(p_ref, o_ref):
    o_ref[...] = p_ref[0] + p_ref[1]


def _combine(partials, n_nodes, d_feat):
    rows = 1000
    n_blk = n_nodes // rows
    return pl.pallas_call(
        _add_body,
        grid=(n_blk,),
        in_specs=[pl.BlockSpec((NC, rows, d_feat), lambda i: (0, i, 0))],
        out_specs=pl.BlockSpec((rows, d_feat), lambda i: (i, 0)),
        out_shape=jax.ShapeDtypeStruct((n_nodes, d_feat), jnp.float32),
    )(partials)


def kernel(src_emb, edge_index, src_emb_in):
    n_edges, d_feat = src_emb.shape
    n_nodes = src_emb_in.shape[0]
    e_per_w = n_edges // (NC * NS)
    dst4 = edge_index.astype(jnp.int32).reshape(
        2, NC * NS, e_per_w // CHUNK, CHUNK
    )
    partials = _sc_partials(n_nodes, n_edges, d_feat)(src_emb, dst4)
    return _combine(partials, n_nodes, d_feat)
